# baseline (speedup 1.0000x reference)
import functools

import jax
import jax.numpy as jnp
import numpy as np
from jax.experimental import pallas as pl
from jax.experimental.pallas import tpu as pltpu

K = 5
LOG31 = float(np.log(31.0))
NS = [12500, 3125, 780, 195]
ES = [200000, 50000, 12500, 3125]
CH = [(1, 32), (32, 64), (64, 128), (128, 256)]


def _logcart(d):
    return jnp.clip(0.5 + 0.5 * jnp.sign(d) * jnp.log1p(30.0 * jnp.abs(d)) / LOG31, 0.0, 1.0)


def _head_kernel(xv_ref, fc1w_ref, fc1b_ref, fc2w_ref, fc2b_ref, out_ref):
    h = xv_ref[...].reshape(1, 8 * 256)
    h = h @ fc1w_ref[...] + fc1b_ref[...][None, :]
    h = jnp.where(h > 0, h, jnp.exp(jnp.minimum(h, 0.0)) - 1.0)
    o = h @ fc2w_ref[...] + fc2b_ref[...][None, :]
    out_ref[...] = jax.nn.log_softmax(o, axis=1)


def _head(xv, fc1_w, fc1_b, fc2_w, fc2_b):
    return pl.pallas_call(
        _head_kernel,
        out_shape=jax.ShapeDtypeStruct((1, 10), jnp.float32),
    )(xv, fc1_w, fc1_b, fc2_w, fc2_b)


def kernel(x, pos, edge_index, cluster1, cluster2, cluster3, cluster4,
           W1, R1, b1, W2, R2, b2, W3, R3, b3, W4, R4, b4,
           fc1_w, fc1_b, fc2_w, fc2_b):
    clusters = [cluster1, cluster2, cluster3, cluster4]
    Ws = [(W1, R1, b1), (W2, R2, b2), (W3, R3, b3), (W4, R4, b4)]
    e = edge_index
    for i in range(4):
        c = clusters[i]
        n = NS[i]
        xp = jax.ops.segment_max(x, c, num_segments=n)
        xp = jnp.where(jnp.isfinite(xp), xp, 0.0)
        cnt = jax.ops.segment_sum(jnp.ones((c.shape[0],), x.dtype), c, num_segments=n)
        posp = jax.ops.segment_sum(pos, c, num_segments=n) / jnp.maximum(cnt, 1.0)[:, None]
        x, pos = xp, posp
        e = c[e[:, :ES[i]]]
        src, dst = e[0], e[1]
        u = _logcart(pos[dst] - pos[src])
        t = u * (K - 1)
        frac = t - jnp.floor(t)
        s = jnp.prod(1.0 - frac, axis=1) + jnp.prod(frac, axis=1)
        W, R, b = Ws[i]
        deg = jax.ops.segment_sum(jnp.ones(dst.shape, x.dtype), dst, num_segments=n)
        if i == 0:
            a = jax.ops.segment_sum(s * x[src, 0], dst, num_segments=n)
            agg = (a / jnp.maximum(deg, 1.0))[:, None] * W[0][None, :]
        else:
            h = x @ W
            msg = s[:, None] * h[src]
            agg = jax.ops.segment_sum(msg, dst, num_segments=n)
            agg = agg / jnp.maximum(deg, 1.0)[:, None]
        x = jax.nn.elu(agg + x @ R + b)
    vid = jnp.clip(jnp.floor(pos * 2.0), 0, 1).astype(jnp.int32)
    vox = vid[:, 0] * 4 + vid[:, 1] * 2 + vid[:, 2]
    xv = jax.ops.segment_max(x, vox, num_segments=8)
    xv = jnp.where(jnp.isfinite(xv), xv, 0.0)
    return _head(xv, fc1_w, fc1_b, fc2_w, fc2_b)


# speedup vs baseline: 1.2996x; 1.2996x over previous
import functools

import jax
import jax.numpy as jnp
import numpy as np
from jax.experimental import pallas as pl
from jax.experimental.pallas import tpu as pltpu

K = 5
LOG31 = float(np.log(31.0))
NS = [12500, 3125, 780, 195]
ES = [200000, 50000, 12500, 3125]
CH = [(1, 32), (32, 64), (64, 128), (128, 256)]


def _logcart(d):
    return jnp.clip(0.5 + 0.5 * jnp.sign(d) * jnp.log1p(30.0 * jnp.abs(d)) / LOG31, 0.0, 1.0)


def _head_kernel(xv_ref, fc1w_ref, fc1b_ref, fc2w_ref, fc2b_ref, out_ref):
    h = xv_ref[...].reshape(1, 8 * 256)
    h = h @ fc1w_ref[...] + fc1b_ref[...][None, :]
    h = jnp.where(h > 0, h, jnp.exp(jnp.minimum(h, 0.0)) - 1.0)
    o = h @ fc2w_ref[...] + fc2b_ref[...][None, :]
    out_ref[...] = jax.nn.log_softmax(o, axis=1)


def _head(xv, fc1_w, fc1_b, fc2_w, fc2_b):
    return pl.pallas_call(
        _head_kernel,
        out_shape=jax.ShapeDtypeStruct((1, 10), jnp.float32),
    )(xv, fc1_w, fc1_b, fc2_w, fc2_b)


def kernel(x, pos, edge_index, cluster1, cluster2, cluster3, cluster4,
           W1, R1, b1, W2, R2, b2, W3, R3, b3, W4, R4, b4,
           fc1_w, fc1_b, fc2_w, fc2_b):
    clusters = [cluster1, cluster2, cluster3, cluster4]
    Ws = [(W1, R1, b1), (W2, R2, b2), (W3, R3, b3), (W4, R4, b4)]
    e = edge_index
    for i in range(4):
        c = clusters[i]
        n = NS[i]
        xp = jax.ops.segment_max(x, c, num_segments=n)
        xp = jnp.where(jnp.isfinite(xp), xp, 0.0)
        cnt = jax.ops.segment_sum(jnp.ones((c.shape[0],), x.dtype), c, num_segments=n)
        posp = jax.ops.segment_sum(pos, c, num_segments=n) / jnp.maximum(cnt, 1.0)[:, None]
        x, pos = xp, posp
        e = c[e[:, :ES[i]]]
        src, dst = e[0], e[1]
        u = _logcart(pos[dst] - pos[src])
        t = u * (K - 1)
        frac = t - jnp.floor(t)
        s = jnp.prod(1.0 - frac, axis=1) + jnp.prod(frac, axis=1)
        W, R, b = Ws[i]
        deg = jax.ops.segment_sum(jnp.ones(dst.shape, x.dtype), dst, num_segments=n)
        h = x @ W
        msg = s[:, None] * h[src]
        agg = jax.ops.segment_sum(msg, dst, num_segments=n)
        agg = agg / jnp.maximum(deg, 1.0)[:, None]
        x = jax.nn.elu(agg + x @ R + b)
    vid = jnp.clip(jnp.floor(pos * 2.0), 0, 1).astype(jnp.int32)
    vox = vid[:, 0] * 4 + vid[:, 1] * 2 + vid[:, 2]
    xv = jax.ops.segment_max(x, vox, num_segments=8)
    xv = jnp.where(jnp.isfinite(xv), xv, 0.0)
    return _head(xv, fc1_w, fc1_b, fc2_w, fc2_b)


# speedup vs baseline: 7.4078x; 5.7002x over previous
import functools

import jax
import jax.numpy as jnp
import numpy as np
from jax import lax
from jax.experimental import pallas as pl
from jax.experimental.pallas import tpu as pltpu
from jax.experimental.pallas import tpu_sc as plsc

K = 5
LOG31 = float(np.log(31.0))
LN2 = float(np.log(2.0))
NS = [12500, 3125, 780, 195]
ES = [200000, 50000, 12500, 3125]
CH = [(1, 32), (32, 64), (64, 128), (128, 256)]
N0 = 50000

NW = 32
CHUNK = 128
SWEEP = NW * CHUNK


def _cdiv(a, b):
    return (a + b - 1) // b


def _pad_to(v, m):
    return _cdiv(v, m) * m


EPAD = [_pad_to(e, SWEEP) for e in ES]
NPAD = [_pad_to(n, CHUNK) for n in NS]


def _log1p30(absd):
    z = 1.0 + 30.0 * absd
    bits = lax.bitcast_convert_type(z, jnp.int32)
    k = lax.shift_right_logical(bits, 23) - 127
    m = lax.bitcast_convert_type(
        lax.bitwise_or(lax.bitwise_and(bits, 0x007FFFFF), 0x3F800000),
        jnp.float32)
    t = (m - 1.0) / (m + 1.0)
    t2 = t * t
    lnm = t * (2.0 + t2 * (2.0 / 3.0 + t2 * (2.0 / 5.0 + t2 * (2.0 / 7.0 + t2 * (2.0 / 9.0)))))
    return k.astype(jnp.float32) * LN2 + lnm


def _basis(dx, dy, dz):
    s0 = jnp.float32(1.0)
    s1 = jnp.float32(1.0)
    for d in (dx, dy, dz):
        sgn = jnp.where(d < 0.0, -1.0, jnp.where(d > 0.0, 1.0, 0.0))
        u = 0.5 + 0.5 * sgn * _log1p30(jnp.abs(d)) / LOG31
        u = jnp.minimum(jnp.maximum(u, 0.0), 1.0)
        t = u * (K - 1)
        fl = t.astype(jnp.int32).astype(jnp.float32)
        frac = t - fl
        s0 = s0 * (1.0 - frac)
        s1 = s1 * frac
    return s0 + s1


def _iota16():
    return lax.iota(jnp.int32, 16)


def _seg_accum(keys, vals, acc_refs, kbuf, vbuf):
    lanes = _iota16()
    sk, perm = plsc.sort_key_val(keys, lanes)
    kbuf[...] = sk
    pvals = []
    for v in vals:
        vbuf[...] = v
        pvals.append(plsc.load_gather(vbuf, [perm]))
    for st in (1, 2, 4, 8):
        idx = jnp.maximum(lanes - st, 0)
        kb = plsc.load_gather(kbuf, [idx])
        samek = jnp.logical_and(kb == sk, lanes >= st)
        for j in range(len(pvals)):
            vbuf[...] = pvals[j]
            vb = plsc.load_gather(vbuf, [idx])
            pvals[j] = pvals[j] + jnp.where(samek, vb, 0.0)
    knext = plsc.load_gather(kbuf, [jnp.minimum(lanes + 1, 15)])
    last = jnp.logical_or(lanes == 15, knext != sk)
    for ref, v in zip(acc_refs, pvals):
        plsc.addupdate_scatter(ref, [sk], v, mask=last)


def _edge_l0(eidx, clpacked, xp, px, py, pz):
    n, e_true, e_pad, npad = NS[0], ES[0], EPAD[0], NPAD[0]
    nhalf = clpacked.shape[0]
    cpt = e_pad // SWEEP
    mesh = plsc.VectorSubcoreMesh(core_axis_name="c", subcore_axis_name="s")

    @functools.partial(
        pl.kernel, mesh=mesh,
        compiler_params=pltpu.CompilerParams(needs_layout_passes=False),
        out_type=[jax.ShapeDtypeStruct((NW, npad), jnp.float32),
                  jax.ShapeDtypeStruct((NW, npad), jnp.float32),
                  jax.ShapeDtypeStruct((2, e_pad), jnp.int32)],
        scratch_types=[
            pltpu.VMEM((nhalf,), jnp.int32),
            pltpu.VMEM((n,), jnp.float32),
            pltpu.VMEM((n,), jnp.float32),
            pltpu.VMEM((n,), jnp.float32),
            pltpu.VMEM((n,), jnp.float32),
            pltpu.VMEM((CHUNK,), jnp.int32),
            pltpu.VMEM((CHUNK,), jnp.int32),
            pltpu.VMEM((CHUNK,), jnp.int32),
            pltpu.VMEM((CHUNK,), jnp.int32),
            pltpu.VMEM((npad,), jnp.float32),
            pltpu.VMEM((npad,), jnp.float32),
            pltpu.VMEM((16,), jnp.int32),
            pltpu.VMEM((16,), jnp.float32),
        ])
    def body(eidx_h, cl_h, x_h, px_h, py_h, pz_h, asum_h, deg_h, emap_h,
             cl_v, x_v, px_v, py_v, pz_v, rs_v, rd_v, ms_v, md_v,
             acc_a, acc_d, kbuf, vbuf):
        cid = lax.axis_index("c")
        sid = lax.axis_index("s")
        wid = sid * 2 + cid
        pltpu.sync_copy(cl_h, cl_v)
        pltpu.sync_copy(x_h, x_v)
        pltpu.sync_copy(px_h, px_v)
        pltpu.sync_copy(py_h, py_v)
        pltpu.sync_copy(pz_h, pz_v)
        zero16 = jnp.zeros((16,), jnp.float32)
        def zacc(r, _):
            acc_a[pl.ds(r * 16, 16)] = zero16
            acc_d[pl.ds(r * 16, 16)] = zero16
            return 0
        lax.fori_loop(0, npad // 16, zacc, 0)

        def chunk(j, _):
            base = j * SWEEP + wid * CHUNK
            pltpu.sync_copy(eidx_h.at[0, pl.ds(base, CHUNK)], rs_v)
            pltpu.sync_copy(eidx_h.at[1, pl.ds(base, CHUNK)], rd_v)
            for g in range(CHUNK // 16):
                sl = pl.ds(g * 16, 16)
                raws = rs_v[sl]
                rawd = rd_v[sl]
                ws = plsc.load_gather(cl_v, [lax.shift_right_logical(raws, 1)])
                wd = plsc.load_gather(cl_v, [lax.shift_right_logical(rawd, 1)])
                sv = lax.bitwise_and(
                    lax.shift_right_logical(ws, lax.bitwise_and(raws, 1) * 16), 0xFFFF)
                dv = lax.bitwise_and(
                    lax.shift_right_logical(wd, lax.bitwise_and(rawd, 1) * 16), 0xFFFF)
                ms_v[sl] = sv
                md_v[sl] = dv
                dx = plsc.load_gather(px_v, [dv]) - plsc.load_gather(px_v, [sv])
                dy = plsc.load_gather(py_v, [dv]) - plsc.load_gather(py_v, [sv])
                dz = plsc.load_gather(pz_v, [dv]) - plsc.load_gather(pz_v, [sv])
                s = _basis(dx, dy, dz)
                validf = jnp.where(base + g * 16 + _iota16() < e_true, 1.0, 0.0)
                a = s * plsc.load_gather(x_v, [sv]) * validf
                _seg_accum(dv, [a, validf], [acc_a, acc_d], kbuf, vbuf)
            pltpu.sync_copy(ms_v, emap_h.at[0, pl.ds(base, CHUNK)])
            pltpu.sync_copy(md_v, emap_h.at[1, pl.ds(base, CHUNK)])
            return 0
        lax.fori_loop(0, cpt, chunk, 0)
        pltpu.sync_copy(acc_a, asum_h.at[wid])
        pltpu.sync_copy(acc_d, deg_h.at[wid])

    return body(eidx, clpacked, xp, px, py, pz)


def _edge_lvl(i, eprev, cl, h, px, py, pz):
    n, n_in, e_true, e_pad, npad = NS[i], NS[i - 1], ES[i], EPAD[i], NPAD[i]
    cout = CH[i][1]
    nrep = _cdiv(cout, 128)
    cpt = e_pad // SWEEP
    mesh = plsc.VectorSubcoreMesh(core_axis_name="c", subcore_axis_name="s")

    @functools.partial(
        pl.kernel, mesh=mesh,
        compiler_params=pltpu.CompilerParams(needs_layout_passes=False),
        out_type=[jax.ShapeDtypeStruct((2, nrep * npad, 128), jnp.float32),
                  jax.ShapeDtypeStruct((NW, npad), jnp.float32),
                  jax.ShapeDtypeStruct((2, e_pad), jnp.int32)],
        scratch_types=[
            pltpu.VMEM((n_in,), jnp.int32),
            pltpu.VMEM((n,), jnp.float32),
            pltpu.VMEM((n,), jnp.float32),
            pltpu.VMEM((n,), jnp.float32),
            pltpu.VMEM((CHUNK,), jnp.int32),
            pltpu.VMEM((CHUNK,), jnp.int32),
            pltpu.VMEM((CHUNK,), jnp.int32),
            pltpu.VMEM((CHUNK,), jnp.int32),
            pltpu.VMEM((CHUNK,), jnp.int32),
            pltpu.VMEM((CHUNK,), jnp.int32),
            pltpu.VMEM((CHUNK,), jnp.float32),
            pltpu.VMEM((CHUNK, 128), jnp.float32),
            pltpu.VMEM((npad,), jnp.float32),
            pltpu.VMEM((16,), jnp.int32),
            pltpu.VMEM((16,), jnp.float32),
            pltpu.VMEM_SHARED((nrep * npad, 128), jnp.float32),
            pltpu.SemaphoreType.DMA,
        ])
    def body(eprev_h, cl_h, h_h, px_h, py_h, pz_h, sums_h, deg_h, emap_h,
             cl_v, px_v, py_v, pz_v, rs_v, rd_v, ms_v, md_v, msj_v, mdj_v,
             s_v, rows_v, acc_d, kbuf, vbuf, acc_sh, sem):
        cid = lax.axis_index("c")
        sid = lax.axis_index("s")
        wid = sid * 2 + cid
        pltpu.sync_copy(cl_h, cl_v)
        pltpu.sync_copy(px_h, px_v)
        pltpu.sync_copy(py_h, py_v)
        pltpu.sync_copy(pz_h, pz_v)
        zero16 = jnp.zeros((16,), jnp.float32)
        def zrow(r, _):
            for c in range(128 // 16):
                rows_v[r, pl.ds(c * 16, 16)] = zero16
            return 0
        lax.fori_loop(0, CHUNK, zrow, 0)
        def zacc(r, _):
            acc_d[pl.ds(r * 16, 16)] = zero16
            return 0
        lax.fori_loop(0, npad // 16, zacc, 0)
        for rc in range(nrep * npad // CHUNK):
            @pl.when(sid == rc % 16)
            def _():
                pltpu.sync_copy(rows_v, acc_sh.at[pl.ds(rc * CHUNK, CHUNK)])
        plsc.subcore_barrier()

        def chunk(j, _):
            base = j * SWEEP + wid * CHUNK
            pltpu.sync_copy(eprev_h.at[0, pl.ds(base, CHUNK)], rs_v)
            pltpu.sync_copy(eprev_h.at[1, pl.ds(base, CHUNK)], rd_v)
            for g in range(CHUNK // 16):
                sl = pl.ds(g * 16, 16)
                sv = plsc.load_gather(cl_v, [rs_v[sl]])
                dv = plsc.load_gather(cl_v, [rd_v[sl]])
                ms_v[sl] = sv
                md_v[sl] = dv
                dx = plsc.load_gather(px_v, [dv]) - plsc.load_gather(px_v, [sv])
                dy = plsc.load_gather(py_v, [dv]) - plsc.load_gather(py_v, [sv])
                dz = plsc.load_gather(pz_v, [dv]) - plsc.load_gather(pz_v, [sv])
                s = _basis(dx, dy, dz)
                validf = jnp.where(base + g * 16 + _iota16() < e_true, 1.0, 0.0)
                s_v[sl] = s * validf
                _seg_accum(dv, [validf], [acc_d], kbuf, vbuf)
            for j in range(nrep):
                if nrep > 1:
                    def scl_idx(g, _):
                        sl = pl.ds(g * 16, 16)
                        msj_v[sl] = ms_v[sl] * nrep + j
                        mdj_v[sl] = md_v[sl] * nrep + j
                        return 0
                    lax.fori_loop(0, CHUNK // 16, scl_idx, 0)
                    src_idx, dst_idx = msj_v, mdj_v
                else:
                    src_idx, dst_idx = ms_v, md_v
                pltpu.async_copy(h_h.at[src_idx], rows_v, sem).wait()
                def scale(g, _):
                    sg = s_v[pl.ds(g * 16, 16)]
                    for lane in range(16):
                        sval = sg[lane]
                        r = g * 16 + lane
                        for c in range(128 // 16):
                            csl = pl.ds(c * 16, 16)
                            rows_v[r, csl] = rows_v[r, csl] * sval
                    return 0
                lax.fori_loop(0, CHUNK // 16, scale, 0)
                pltpu.sync_copy(rows_v, acc_sh.at[dst_idx], add=True)
            pltpu.sync_copy(ms_v, emap_h.at[0, pl.ds(base, CHUNK)])
            pltpu.sync_copy(md_v, emap_h.at[1, pl.ds(base, CHUNK)])
            return 0
        lax.fori_loop(0, cpt, chunk, 0)
        plsc.subcore_barrier()
        for rc in range(nrep * npad // CHUNK):
            @pl.when(sid == rc % 16)
            def _():
                pltpu.sync_copy(acc_sh.at[pl.ds(rc * CHUNK, CHUNK)],
                                sums_h.at[cid, pl.ds(rc * CHUNK, CHUNK)])
        pltpu.sync_copy(acc_d, deg_h.at[wid])

    return body(eprev, cl, h, px, py, pz)


def _elu(v):
    return jnp.where(v > 0, v, jnp.exp(jnp.minimum(v, 0.0)) - 1.0)


def _combine0_body(asum_ref, degs_ref, xp_ref, w_ref, r_ref, b_ref, out_ref):
    a = jnp.sum(asum_ref[...], axis=0)[:NS[0]]
    deg = jnp.sum(degs_ref[...], axis=0)[:NS[0]]
    agg = (a / jnp.maximum(deg, 1.0))[:, None] * w_ref[0][None, :]
    root = xp_ref[...] @ r_ref[...]
    out_ref[...] = _elu(agg + root + b_ref[...][None, :])


def _combine0(asum, degs, xp, W1, R1, b1):
    return pl.pallas_call(
        _combine0_body,
        out_shape=jax.ShapeDtypeStruct((NS[0], CH[0][1]), jnp.float32),
    )(asum, degs, xp, W1, R1, b1)


def _prep_body(nrep, x_ref, w_ref, rw_ref, b_ref, h_ref, r_ref):
    h = x_ref[...] @ w_ref[...]
    n, cout = h.shape
    if cout < 128:
        h = jnp.concatenate(
            [h, jnp.zeros((n, 128 - cout), jnp.float32)], axis=1)
    h_ref[...] = h.reshape(nrep * n, 128)
    r_ref[...] = x_ref[...] @ rw_ref[...] + b_ref[...][None, :]


def _prep(i, x, W, R, b):
    n, cout = NS[i], CH[i][1]
    nrep = _cdiv(cout, 128)
    return pl.pallas_call(
        functools.partial(_prep_body, nrep),
        out_shape=[jax.ShapeDtypeStruct((nrep * n, 128), jnp.float32),
                   jax.ShapeDtypeStruct((n, cout), jnp.float32)],
    )(x, W, R, b)


def _combine_body(n, cout, npad, sums_ref, degs_ref, r_ref, out_ref):
    nrep = _cdiv(cout, 128)
    acc = (sums_ref[0] + sums_ref[1]).reshape(npad, nrep * 128)
    deg = jnp.sum(degs_ref[...], axis=0)
    agg = acc[:n, :cout] / jnp.maximum(deg[:n], 1.0)[:, None]
    out_ref[...] = _elu(agg + r_ref[...])


def _combine(i, sums, degs, r):
    n, cout = NS[i], CH[i][1]
    return pl.pallas_call(
        functools.partial(_combine_body, n, cout, NPAD[i]),
        out_shape=jax.ShapeDtypeStruct((n, cout), jnp.float32),
    )(sums, degs, r)


def _head_body(x_ref, pos_ref, fc1w_ref, fc1b_ref, fc2w_ref, fc2b_ref, out_ref):
    pos = pos_ref[...]
    vid = jnp.clip(jnp.floor(pos * 2.0), 0, 1).astype(jnp.int32)
    vox = vid[:, 0] * 4 + vid[:, 1] * 2 + vid[:, 2]
    x = x_ref[...]
    cells = []
    for k in range(8):
        m = (vox == k)[:, None]
        cells.append(jnp.max(jnp.where(m, x, -jnp.inf), axis=0))
    xv = jnp.stack(cells, axis=0)
    xv = jnp.where(jnp.isfinite(xv), xv, 0.0)
    hidden = xv.reshape(1, 8 * 256) @ fc1w_ref[...] + fc1b_ref[...][None, :]
    hidden = _elu(hidden)
    o = hidden @ fc2w_ref[...] + fc2b_ref[...][None, :]
    out_ref[...] = jax.nn.log_softmax(o, axis=1)


def _head(x, pos, fc1_w, fc1_b, fc2_w, fc2_b):
    return pl.pallas_call(
        _head_body,
        out_shape=jax.ShapeDtypeStruct((1, 10), jnp.float32),
    )(x, pos, fc1_w, fc1_b, fc2_w, fc2_b)


def kernel(x, pos, edge_index, cluster1, cluster2, cluster3, cluster4,
           W1, R1, b1, W2, R2, b2, W3, R3, b3, W4, R4, b4,
           fc1_w, fc1_b, fc2_w, fc2_b):
    clusters = [cluster1, cluster2, cluster3, cluster4]
    Ws = [(W1, R1, b1), (W2, R2, b2), (W3, R3, b3), (W4, R4, b4)]
    e = edge_index
    for i in range(4):
        c = clusters[i]
        n = NS[i]
        xp = jax.ops.segment_max(x, c, num_segments=n)
        xp = jnp.where(jnp.isfinite(xp), xp, 0.0)
        cnt = jax.ops.segment_sum(jnp.ones((c.shape[0],), x.dtype), c, num_segments=n)
        posp = jax.ops.segment_sum(pos, c, num_segments=n) / jnp.maximum(cnt, 1.0)[:, None]
        px, py, pz = posp[:, 0], posp[:, 1], posp[:, 2]
        W, R, b = Ws[i]
        if i == 0:
            cu = c.astype(jnp.uint32)
            clp = (cu[0::2] | (cu[1::2] << 16)).astype(jnp.int32)
            asum, degs, e = _edge_l0(e, clp, xp[:, 0], px, py, pz)
            x = _combine0(asum, degs, xp, W, R, b)
        else:
            h, r = _prep(i, xp, W, R, b)
            sums, degs, e = _edge_lvl(i, e, c, h, px, py, pz)
            x = _combine(i, sums, degs, r)
        pos = posp
    return _head(x, pos, fc1_w, fc1_b, fc2_w, fc2_b)


# speedup vs baseline: 12.9566x; 1.7490x over previous
import functools

import jax
import jax.numpy as jnp
import numpy as np
from jax import lax
from jax.experimental import pallas as pl
from jax.experimental.pallas import tpu as pltpu
from jax.experimental.pallas import tpu_sc as plsc

K = 5
LOG31 = float(np.log(31.0))
LN2 = float(np.log(2.0))
NS = [12500, 3125, 780, 195]
ES = [200000, 50000, 12500, 3125]
CH = [(1, 32), (32, 64), (64, 128), (128, 256)]
N0 = 50000

NW = 32
CHUNK = 128
SWEEP = NW * CHUNK


def _cdiv(a, b):
    return (a + b - 1) // b


def _pad_to(v, m):
    return _cdiv(v, m) * m


EPAD = [_pad_to(e, SWEEP) for e in ES]
NPAD = [_pad_to(n, CHUNK) for n in NS]


def _log1p30(absd):
    z = 1.0 + 30.0 * absd
    bits = lax.bitcast_convert_type(z, jnp.int32)
    k = lax.shift_right_logical(bits, 23) - 127
    m = lax.bitcast_convert_type(
        lax.bitwise_or(lax.bitwise_and(bits, 0x007FFFFF), 0x3F800000),
        jnp.float32)
    t = (m - 1.0) / (m + 1.0)
    t2 = t * t
    lnm = t * (2.0 + t2 * (2.0 / 3.0 + t2 * (2.0 / 5.0 + t2 * (2.0 / 7.0 + t2 * (2.0 / 9.0)))))
    return k.astype(jnp.float32) * LN2 + lnm


def _basis(dx, dy, dz):
    s0 = jnp.float32(1.0)
    s1 = jnp.float32(1.0)
    for d in (dx, dy, dz):
        sgn = jnp.where(d < 0.0, -1.0, jnp.where(d > 0.0, 1.0, 0.0))
        u = 0.5 + 0.5 * sgn * _log1p30(jnp.abs(d)) / LOG31
        u = jnp.minimum(jnp.maximum(u, 0.0), 1.0)
        t = u * (K - 1)
        fl = t.astype(jnp.int32).astype(jnp.float32)
        frac = t - fl
        s0 = s0 * (1.0 - frac)
        s1 = s1 * frac
    return s0 + s1


def _iota16():
    return lax.iota(jnp.int32, 16)


def _seg_accum(keys, vals, acc_refs, kbuf, vbuf, ops=None):
    if ops is None:
        ops = ["add"] * len(vals)
    lanes = _iota16()
    sk, perm = plsc.sort_key_val(keys, lanes)
    kbuf[...] = sk
    pvals = []
    for v in vals:
        vbuf[...] = v
        pvals.append(plsc.load_gather(vbuf, [perm]))
    for st in (1, 2, 4, 8):
        idx = jnp.maximum(lanes - st, 0)
        kb = plsc.load_gather(kbuf, [idx])
        samek = jnp.logical_and(kb == sk, lanes >= st)
        for j, op in enumerate(ops):
            vbuf[...] = pvals[j]
            vb = plsc.load_gather(vbuf, [idx])
            if op == "add":
                pvals[j] = pvals[j] + jnp.where(samek, vb, 0.0)
            else:
                pvals[j] = jnp.maximum(pvals[j], jnp.where(samek, vb, -jnp.inf))
    knext = plsc.load_gather(kbuf, [jnp.minimum(lanes + 1, 15)])
    last = jnp.logical_or(lanes == 15, knext != sk)
    for ref, v, op in zip(acc_refs, pvals, ops):
        if op == "add":
            plsc.addupdate_scatter(ref, [sk], v, mask=last)
        else:
            old = plsc.load_gather(ref, [sk])
            plsc.store_scatter(ref, [sk], jnp.maximum(old, v), mask=last)


NP = 8
PSWEEP = NP * CHUNK
NINPAD = [_pad_to(v, PSWEEP) for v in (N0, NS[0], NS[1], NS[2])]


def _pool_l0(cl, x, px, py, pz):
    npad = NPAD[0]
    cpt = NINPAD[0] // PSWEEP
    mesh = plsc.VectorSubcoreMesh(core_axis_name="c", subcore_axis_name="s")

    @functools.partial(
        pl.kernel, mesh=mesh,
        compiler_params=pltpu.CompilerParams(needs_layout_passes=False),
        out_type=[jax.ShapeDtypeStruct((NP, npad), jnp.float32)
                  for _ in range(5)],
        scratch_types=[
            pltpu.VMEM((CHUNK,), jnp.int32),
            pltpu.VMEM((CHUNK,), jnp.float32),
            pltpu.VMEM((CHUNK,), jnp.float32),
            pltpu.VMEM((CHUNK,), jnp.float32),
            pltpu.VMEM((CHUNK,), jnp.float32),
            pltpu.VMEM((npad,), jnp.float32),
            pltpu.VMEM((npad,), jnp.float32),
            pltpu.VMEM((npad,), jnp.float32),
            pltpu.VMEM((npad,), jnp.float32),
            pltpu.VMEM((npad,), jnp.float32),
            pltpu.VMEM((16,), jnp.int32),
            pltpu.VMEM((16,), jnp.float32),
        ])
    def body(cl_h, x_h, px_h, py_h, pz_h, xm_h, sx_h, sy_h, sz_h, ct_h,
             cl_v, x_v, px_v, py_v, pz_v, am, ax, ay, az, ac, kbuf, vbuf):
        cid = lax.axis_index("c")
        sid = lax.axis_index("s")
        wid = sid * 2 + cid

        @pl.when(wid < NP)
        def _():
            zero16 = jnp.zeros((16,), jnp.float32)
            ninf16 = jnp.full((16,), -jnp.inf, jnp.float32)
            def zacc(r, _):
                sl = pl.ds(r * 16, 16)
                am[sl] = ninf16
                ax[sl] = zero16
                ay[sl] = zero16
                az[sl] = zero16
                ac[sl] = zero16
                return 0
            lax.fori_loop(0, npad // 16, zacc, 0)

            def chunk(j, _):
                base = j * PSWEEP + wid * CHUNK
                pltpu.sync_copy(cl_h.at[pl.ds(base, CHUNK)], cl_v)
                pltpu.sync_copy(x_h.at[pl.ds(base, CHUNK)], x_v)
                pltpu.sync_copy(px_h.at[pl.ds(base, CHUNK)], px_v)
                pltpu.sync_copy(py_h.at[pl.ds(base, CHUNK)], py_v)
                pltpu.sync_copy(pz_h.at[pl.ds(base, CHUNK)], pz_v)
                for g in range(CHUNK // 16):
                    sl = pl.ds(g * 16, 16)
                    keys = cl_v[sl]
                    one = jnp.ones((16,), jnp.float32)
                    _seg_accum(keys, [x_v[sl], px_v[sl], py_v[sl], pz_v[sl], one],
                               [am, ax, ay, az, ac], kbuf, vbuf,
                               ops=["max", "add", "add", "add", "add"])
                return 0
            lax.fori_loop(0, cpt, chunk, 0)
            pltpu.sync_copy(am, xm_h.at[wid])
            pltpu.sync_copy(ax, sx_h.at[wid])
            pltpu.sync_copy(ay, sy_h.at[wid])
            pltpu.sync_copy(az, sz_h.at[wid])
            pltpu.sync_copy(ac, ct_h.at[wid])

    return body(cl, x, px, py, pz)


def _pool_lvl(i, cl, x, px, py, pz):
    npad = NPAD[i]
    cin = CH[i][0]
    cpt = NINPAD[i] // PSWEEP
    mesh = plsc.VectorSubcoreMesh(core_axis_name="c", subcore_axis_name="s")

    @functools.partial(
        pl.kernel, mesh=mesh,
        compiler_params=pltpu.CompilerParams(needs_layout_passes=False),
        out_type=[jax.ShapeDtypeStruct((NP, npad * cin), jnp.float32)]
                 + [jax.ShapeDtypeStruct((NP, npad), jnp.float32)
                    for _ in range(4)],
        scratch_types=[
            pltpu.VMEM((CHUNK,), jnp.int32),
            pltpu.VMEM((CHUNK * cin,), jnp.float32),
            pltpu.VMEM((CHUNK,), jnp.float32),
            pltpu.VMEM((CHUNK,), jnp.float32),
            pltpu.VMEM((CHUNK,), jnp.float32),
            pltpu.VMEM((npad * cin,), jnp.float32),
            pltpu.VMEM((npad,), jnp.float32),
            pltpu.VMEM((npad,), jnp.float32),
            pltpu.VMEM((npad,), jnp.float32),
            pltpu.VMEM((npad,), jnp.float32),
            pltpu.VMEM((16,), jnp.int32),
            pltpu.VMEM((16,), jnp.float32),
        ])
    def body(cl_h, x_h, px_h, py_h, pz_h, xm_h, sx_h, sy_h, sz_h, ct_h,
             cl_v, xr_v, px_v, py_v, pz_v, am, ax, ay, az, ac, kbuf, vbuf):
        cid = lax.axis_index("c")
        sid = lax.axis_index("s")
        wid = sid * 2 + cid

        @pl.when(wid < NP)
        def _():
            zero16 = jnp.zeros((16,), jnp.float32)
            ninf16 = jnp.full((16,), -jnp.inf, jnp.float32)
            def zacc(r, _):
                am[pl.ds(r * 16, 16)] = ninf16
                return 0
            lax.fori_loop(0, npad * cin // 16, zacc, 0)
            def zacc2(r, _):
                sl = pl.ds(r * 16, 16)
                ax[sl] = zero16
                ay[sl] = zero16
                az[sl] = zero16
                ac[sl] = zero16
                return 0
            lax.fori_loop(0, npad // 16, zacc2, 0)

            def chunk(j, _):
                base = j * PSWEEP + wid * CHUNK
                pltpu.sync_copy(cl_h.at[pl.ds(base, CHUNK)], cl_v)
                pltpu.sync_copy(x_h.at[pl.ds(base * cin, CHUNK * cin)], xr_v)
                pltpu.sync_copy(px_h.at[pl.ds(base, CHUNK)], px_v)
                pltpu.sync_copy(py_h.at[pl.ds(base, CHUNK)], py_v)
                pltpu.sync_copy(pz_h.at[pl.ds(base, CHUNK)], pz_v)
                for g in range(CHUNK // 16):
                    sl = pl.ds(g * 16, 16)
                    keys = cl_v[sl]
                    one = jnp.ones((16,), jnp.float32)
                    _seg_accum(keys, [px_v[sl], py_v[sl], pz_v[sl], one],
                               [ax, ay, az, ac], kbuf, vbuf)
                    for lane in range(16):
                        kb = keys[lane] * cin
                        rb = (g * 16 + lane) * cin
                        for c in range(cin // 16):
                            asl = pl.ds(kb + c * 16, 16)
                            xsl = pl.ds(rb + c * 16, 16)
                            am[asl] = jnp.maximum(am[asl], xr_v[xsl])
                return 0
            lax.fori_loop(0, cpt, chunk, 0)
            pltpu.sync_copy(am, xm_h.at[wid])
            pltpu.sync_copy(ax, sx_h.at[wid])
            pltpu.sync_copy(ay, sy_h.at[wid])
            pltpu.sync_copy(az, sz_h.at[wid])
            pltpu.sync_copy(ac, ct_h.at[wid])

    return body(cl, x, px, py, pz)


def _edge_l0(eidx, clpacked, xp, px, py, pz):
    n, e_true, e_pad, npad = NS[0], ES[0], EPAD[0], NPAD[0]
    nhalf = clpacked.shape[0]
    cpt = e_pad // SWEEP
    mesh = plsc.VectorSubcoreMesh(core_axis_name="c", subcore_axis_name="s")

    @functools.partial(
        pl.kernel, mesh=mesh,
        compiler_params=pltpu.CompilerParams(needs_layout_passes=False),
        out_type=[jax.ShapeDtypeStruct((NW, npad), jnp.float32),
                  jax.ShapeDtypeStruct((NW, npad), jnp.float32),
                  jax.ShapeDtypeStruct((2, e_pad), jnp.int32)],
        scratch_types=[
            pltpu.VMEM((nhalf,), jnp.int32),
            pltpu.VMEM((n,), jnp.float32),
            pltpu.VMEM((n,), jnp.float32),
            pltpu.VMEM((n,), jnp.float32),
            pltpu.VMEM((n,), jnp.float32),
            pltpu.VMEM((CHUNK,), jnp.int32),
            pltpu.VMEM((CHUNK,), jnp.int32),
            pltpu.VMEM((CHUNK,), jnp.int32),
            pltpu.VMEM((CHUNK,), jnp.int32),
            pltpu.VMEM((npad,), jnp.float32),
            pltpu.VMEM((npad,), jnp.float32),
            pltpu.VMEM((16,), jnp.int32),
            pltpu.VMEM((16,), jnp.float32),
        ])
    def body(eidx_h, cl_h, x_h, px_h, py_h, pz_h, asum_h, deg_h, emap_h,
             cl_v, x_v, px_v, py_v, pz_v, rs_v, rd_v, ms_v, md_v,
             acc_a, acc_d, kbuf, vbuf):
        cid = lax.axis_index("c")
        sid = lax.axis_index("s")
        wid = sid * 2 + cid
        pltpu.sync_copy(cl_h, cl_v)
        pltpu.sync_copy(x_h, x_v)
        pltpu.sync_copy(px_h, px_v)
        pltpu.sync_copy(py_h, py_v)
        pltpu.sync_copy(pz_h, pz_v)
        zero16 = jnp.zeros((16,), jnp.float32)
        def zacc(r, _):
            acc_a[pl.ds(r * 16, 16)] = zero16
            acc_d[pl.ds(r * 16, 16)] = zero16
            return 0
        lax.fori_loop(0, npad // 16, zacc, 0)

        def chunk(j, _):
            base = j * SWEEP + wid * CHUNK
            pltpu.sync_copy(eidx_h.at[0, pl.ds(base, CHUNK)], rs_v)
            pltpu.sync_copy(eidx_h.at[1, pl.ds(base, CHUNK)], rd_v)
            for g in range(CHUNK // 16):
                sl = pl.ds(g * 16, 16)
                raws = rs_v[sl]
                rawd = rd_v[sl]
                ws = plsc.load_gather(cl_v, [lax.shift_right_logical(raws, 1)])
                wd = plsc.load_gather(cl_v, [lax.shift_right_logical(rawd, 1)])
                sv = lax.bitwise_and(
                    lax.shift_right_logical(ws, lax.bitwise_and(raws, 1) * 16), 0xFFFF)
                dv = lax.bitwise_and(
                    lax.shift_right_logical(wd, lax.bitwise_and(rawd, 1) * 16), 0xFFFF)
                ms_v[sl] = sv
                md_v[sl] = dv
                dx = plsc.load_gather(px_v, [dv]) - plsc.load_gather(px_v, [sv])
                dy = plsc.load_gather(py_v, [dv]) - plsc.load_gather(py_v, [sv])
                dz = plsc.load_gather(pz_v, [dv]) - plsc.load_gather(pz_v, [sv])
                s = _basis(dx, dy, dz)
                validf = jnp.where(base + g * 16 + _iota16() < e_true, 1.0, 0.0)
                a = s * plsc.load_gather(x_v, [sv]) * validf
                _seg_accum(dv, [a, validf], [acc_a, acc_d], kbuf, vbuf)
            pltpu.sync_copy(ms_v, emap_h.at[0, pl.ds(base, CHUNK)])
            pltpu.sync_copy(md_v, emap_h.at[1, pl.ds(base, CHUNK)])
            return 0
        lax.fori_loop(0, cpt, chunk, 0)
        pltpu.sync_copy(acc_a, asum_h.at[wid])
        pltpu.sync_copy(acc_d, deg_h.at[wid])

    return body(eidx, clpacked, xp, px, py, pz)


def _edge_lvl(i, eprev, cl, h, px, py, pz):
    n, n_in, e_true, e_pad, npad = NS[i], NS[i - 1], ES[i], EPAD[i], NPAD[i]
    cout = CH[i][1]
    nrep = _cdiv(cout, 128)
    cpt = e_pad // SWEEP
    mesh = plsc.VectorSubcoreMesh(core_axis_name="c", subcore_axis_name="s")

    @functools.partial(
        pl.kernel, mesh=mesh,
        compiler_params=pltpu.CompilerParams(needs_layout_passes=False),
        out_type=[jax.ShapeDtypeStruct((2, nrep * npad, 128), jnp.float32),
                  jax.ShapeDtypeStruct((NW, npad), jnp.float32),
                  jax.ShapeDtypeStruct((2, e_pad), jnp.int32)],
        scratch_types=[
            pltpu.VMEM((n_in,), jnp.int32),
            pltpu.VMEM((n,), jnp.float32),
            pltpu.VMEM((n,), jnp.float32),
            pltpu.VMEM((n,), jnp.float32),
            pltpu.VMEM((CHUNK,), jnp.int32),
            pltpu.VMEM((CHUNK,), jnp.int32),
            pltpu.VMEM((CHUNK,), jnp.int32),
            pltpu.VMEM((CHUNK,), jnp.int32),
            pltpu.VMEM((CHUNK,), jnp.int32),
            pltpu.VMEM((CHUNK,), jnp.int32),
            pltpu.VMEM((CHUNK,), jnp.float32),
            pltpu.VMEM((CHUNK, 128), jnp.float32),
            pltpu.VMEM((npad,), jnp.float32),
            pltpu.VMEM((16,), jnp.int32),
            pltpu.VMEM((16,), jnp.float32),
            pltpu.VMEM_SHARED((nrep * npad, 128), jnp.float32),
            pltpu.SemaphoreType.DMA,
        ])
    def body(eprev_h, cl_h, h_h, px_h, py_h, pz_h, sums_h, deg_h, emap_h,
             cl_v, px_v, py_v, pz_v, rs_v, rd_v, ms_v, md_v, msj_v, mdj_v,
             s_v, rows_v, acc_d, kbuf, vbuf, acc_sh, sem):
        cid = lax.axis_index("c")
        sid = lax.axis_index("s")
        wid = sid * 2 + cid
        pltpu.sync_copy(cl_h, cl_v)
        pltpu.sync_copy(px_h, px_v)
        pltpu.sync_copy(py_h, py_v)
        pltpu.sync_copy(pz_h, pz_v)
        zero16 = jnp.zeros((16,), jnp.float32)
        def zrow(r, _):
            for c in range(128 // 16):
                rows_v[r, pl.ds(c * 16, 16)] = zero16
            return 0
        lax.fori_loop(0, CHUNK, zrow, 0)
        def zacc(r, _):
            acc_d[pl.ds(r * 16, 16)] = zero16
            return 0
        lax.fori_loop(0, npad // 16, zacc, 0)
        for rc in range(nrep * npad // CHUNK):
            @pl.when(sid == rc % 16)
            def _():
                pltpu.sync_copy(rows_v, acc_sh.at[pl.ds(rc * CHUNK, CHUNK)])
        plsc.subcore_barrier()

        def chunk(j, _):
            base = j * SWEEP + wid * CHUNK
            pltpu.sync_copy(eprev_h.at[0, pl.ds(base, CHUNK)], rs_v)
            pltpu.sync_copy(eprev_h.at[1, pl.ds(base, CHUNK)], rd_v)
            for g in range(CHUNK // 16):
                sl = pl.ds(g * 16, 16)
                sv = plsc.load_gather(cl_v, [rs_v[sl]])
                dv = plsc.load_gather(cl_v, [rd_v[sl]])
                ms_v[sl] = sv
                md_v[sl] = dv
                dx = plsc.load_gather(px_v, [dv]) - plsc.load_gather(px_v, [sv])
                dy = plsc.load_gather(py_v, [dv]) - plsc.load_gather(py_v, [sv])
                dz = plsc.load_gather(pz_v, [dv]) - plsc.load_gather(pz_v, [sv])
                s = _basis(dx, dy, dz)
                validf = jnp.where(base + g * 16 + _iota16() < e_true, 1.0, 0.0)
                s_v[sl] = s * validf
                _seg_accum(dv, [validf], [acc_d], kbuf, vbuf)
            for j in range(nrep):
                if nrep > 1:
                    def scl_idx(g, _):
                        sl = pl.ds(g * 16, 16)
                        msj_v[sl] = ms_v[sl] * nrep + j
                        mdj_v[sl] = md_v[sl] * nrep + j
                        return 0
                    lax.fori_loop(0, CHUNK // 16, scl_idx, 0)
                    src_idx, dst_idx = msj_v, mdj_v
                else:
                    src_idx, dst_idx = ms_v, md_v
                pltpu.async_copy(h_h.at[src_idx], rows_v, sem).wait()
                def scale(g, _):
                    sg = s_v[pl.ds(g * 16, 16)]
                    for lane in range(16):
                        sval = sg[lane]
                        r = g * 16 + lane
                        for c in range(128 // 16):
                            csl = pl.ds(c * 16, 16)
                            rows_v[r, csl] = rows_v[r, csl] * sval
                    return 0
                lax.fori_loop(0, CHUNK // 16, scale, 0)
                pltpu.sync_copy(rows_v, acc_sh.at[dst_idx], add=True)
            pltpu.sync_copy(ms_v, emap_h.at[0, pl.ds(base, CHUNK)])
            pltpu.sync_copy(md_v, emap_h.at[1, pl.ds(base, CHUNK)])
            return 0
        lax.fori_loop(0, cpt, chunk, 0)
        plsc.subcore_barrier()
        for rc in range(nrep * npad // CHUNK):
            @pl.when(sid == rc % 16)
            def _():
                pltpu.sync_copy(acc_sh.at[pl.ds(rc * CHUNK, CHUNK)],
                                sums_h.at[cid, pl.ds(rc * CHUNK, CHUNK)])
        pltpu.sync_copy(acc_d, deg_h.at[wid])

    return body(eprev, cl, h, px, py, pz)


def _elu(v):
    return jnp.where(v > 0, v, jnp.exp(jnp.minimum(v, 0.0)) - 1.0)


def _combine0_body(asum_ref, degs_ref, xp_ref, w_ref, r_ref, b_ref, out_ref):
    a = jnp.sum(asum_ref[...], axis=0)[:NS[0]]
    deg = jnp.sum(degs_ref[...], axis=0)[:NS[0]]
    agg = (a / jnp.maximum(deg, 1.0))[:, None] * w_ref[0][None, :]
    root = xp_ref[...] @ r_ref[...]
    out_ref[...] = _elu(agg + root + b_ref[...][None, :])


def _combine0(asum, degs, xp, W1, R1, b1):
    return pl.pallas_call(
        _combine0_body,
        out_shape=jax.ShapeDtypeStruct((NS[0], CH[0][1]), jnp.float32),
    )(asum, degs, xp, W1, R1, b1)


def _poolcomb0_body(xm_ref, sx_ref, sy_ref, sz_ref, ct_ref,
                    xp_ref, px_ref, py_ref, pz_ref):
    n = NS[0]
    xm = jnp.max(xm_ref[...], axis=0)[:n]
    xp_ref[...] = jnp.where(jnp.isfinite(xm), xm, 0.0)[:, None]
    cnt = jnp.maximum(jnp.sum(ct_ref[...], axis=0)[:n], 1.0)
    px_ref[...] = jnp.sum(sx_ref[...], axis=0)[:n] / cnt
    py_ref[...] = jnp.sum(sy_ref[...], axis=0)[:n] / cnt
    pz_ref[...] = jnp.sum(sz_ref[...], axis=0)[:n] / cnt


def _poolcomb0(xm, sx, sy, sz, ct):
    n = NS[0]
    return pl.pallas_call(
        _poolcomb0_body,
        out_shape=[jax.ShapeDtypeStruct((n, 1), jnp.float32)]
                  + [jax.ShapeDtypeStruct((n,), jnp.float32)] * 3,
    )(xm, sx, sy, sz, ct)


def _prep_pool_body(i, xm_ref, sx_ref, sy_ref, sz_ref, ct_ref,
                    w_ref, rw_ref, b_ref,
                    h_ref, r_ref, px_ref, py_ref, pz_ref):
    n, cout = NS[i], CH[i][1]
    cin = CH[i][0]
    nrep = _cdiv(cout, 128)
    xm = jnp.max(xm_ref[...].reshape(NP, -1, cin), axis=0)[:n]
    xp = jnp.where(jnp.isfinite(xm), xm, 0.0)
    cnt = jnp.maximum(jnp.sum(ct_ref[...], axis=0)[:n], 1.0)
    px_ref[...] = jnp.sum(sx_ref[...], axis=0)[:n] / cnt
    py_ref[...] = jnp.sum(sy_ref[...], axis=0)[:n] / cnt
    pz_ref[...] = jnp.sum(sz_ref[...], axis=0)[:n] / cnt
    h = xp @ w_ref[...]
    if cout < 128:
        h = jnp.concatenate(
            [h, jnp.zeros((n, 128 - cout), jnp.float32)], axis=1)
    h_ref[...] = h.reshape(nrep * n, 128)
    r_ref[...] = xp @ rw_ref[...] + b_ref[...][None, :]


def _prep_pool(i, xm, sx, sy, sz, ct, W, R, b):
    n, cout = NS[i], CH[i][1]
    nrep = _cdiv(cout, 128)
    return pl.pallas_call(
        functools.partial(_prep_pool_body, i),
        out_shape=[jax.ShapeDtypeStruct((nrep * n, 128), jnp.float32),
                   jax.ShapeDtypeStruct((n, cout), jnp.float32)]
                  + [jax.ShapeDtypeStruct((n,), jnp.float32)] * 3,
    )(xm, sx, sy, sz, ct, W, R, b)


def _combine_body(n, cout, npad, sums_ref, degs_ref, r_ref, out_ref):
    nrep = _cdiv(cout, 128)
    acc = (sums_ref[0] + sums_ref[1]).reshape(npad, nrep * 128)
    deg = jnp.sum(degs_ref[...], axis=0)
    agg = acc[:n, :cout] / jnp.maximum(deg[:n], 1.0)[:, None]
    out_ref[...] = _elu(agg + r_ref[...])


def _combine(i, sums, degs, r):
    n, cout = NS[i], CH[i][1]
    return pl.pallas_call(
        functools.partial(_combine_body, n, cout, NPAD[i]),
        out_shape=jax.ShapeDtypeStruct((n, cout), jnp.float32),
    )(sums, degs, r)


def _head_body(x_ref, pos_ref, fc1w_ref, fc1b_ref, fc2w_ref, fc2b_ref, out_ref):
    pos = pos_ref[...]
    vid = jnp.clip(jnp.floor(pos * 2.0), 0, 1).astype(jnp.int32)
    vox = vid[:, 0] * 4 + vid[:, 1] * 2 + vid[:, 2]
    x = x_ref[...]
    cells = []
    for k in range(8):
        m = (vox == k)[:, None]
        cells.append(jnp.max(jnp.where(m, x, -jnp.inf), axis=0))
    xv = jnp.stack(cells, axis=0)
    xv = jnp.where(jnp.isfinite(xv), xv, 0.0)
    hidden = xv.reshape(1, 8 * 256) @ fc1w_ref[...] + fc1b_ref[...][None, :]
    hidden = _elu(hidden)
    o = hidden @ fc2w_ref[...] + fc2b_ref[...][None, :]
    out_ref[...] = jax.nn.log_softmax(o, axis=1)


def _head(x, pos, fc1_w, fc1_b, fc2_w, fc2_b):
    return pl.pallas_call(
        _head_body,
        out_shape=jax.ShapeDtypeStruct((1, 10), jnp.float32),
    )(x, pos, fc1_w, fc1_b, fc2_w, fc2_b)


def _padv(v, ln, val):
    return jnp.concatenate([v, jnp.full((ln - v.shape[0],), val, v.dtype)])


def _padr(m, ln):
    return jnp.concatenate(
        [m, jnp.zeros((ln - m.shape[0], m.shape[1]), m.dtype)], axis=0)


def kernel(x, pos, edge_index, cluster1, cluster2, cluster3, cluster4,
           W1, R1, b1, W2, R2, b2, W3, R3, b3, W4, R4, b4,
           fc1_w, fc1_b, fc2_w, fc2_b):
    clusters = [cluster1, cluster2, cluster3, cluster4]
    Ws = [(W1, R1, b1), (W2, R2, b2), (W3, R3, b3), (W4, R4, b4)]
    e = edge_index

    ninp = NINPAD[0]
    cl0p = _padv(cluster1, ninp, NPAD[0] - 1)
    x0p = _padv(x[:, 0], ninp, 0.0)
    p0 = [_padv(pos[:, d], ninp, 0.0) for d in range(3)]
    xm, sx, sy, sz, ct = _pool_l0(cl0p, x0p, *p0)
    xp1, px, py, pz = _poolcomb0(xm, sx, sy, sz, ct)
    cu = cluster1.astype(jnp.uint32)
    clp = (cu[0::2] | (cu[1::2] << 16)).astype(jnp.int32)
    asum, degs, e = _edge_l0(e, clp, jnp.reshape(xp1, (NS[0],)), px, py, pz)
    xcur = _combine0(asum, degs, xp1, *Ws[0])

    for i in range(1, 4):
        c = clusters[i]
        ninp = NINPAD[i]
        clip_ = _padv(c, ninp, NPAD[i] - 1)
        xpad = _padr(xcur, ninp).reshape(-1)
        ppad = [_padv(q, ninp, 0.0) for q in (px, py, pz)]
        xm, sx, sy, sz, ct = _pool_lvl(i, clip_, xpad, *ppad)
        W, R, b = Ws[i]
        h, r, px, py, pz = _prep_pool(i, xm, sx, sy, sz, ct, W, R, b)
        sums, degs, e = _edge_lvl(i, e, c, h, px, py, pz)
        xcur = _combine(i, sums, degs, r)

    pos4 = jnp.stack([px, py, pz], axis=1)
    return _head(xcur, pos4, fc1_w, fc1_b, fc2_w, fc2_b)


# speedup vs baseline: 15.2988x; 1.1808x over previous
import functools

import jax
import jax.numpy as jnp
import numpy as np
from jax import lax
from jax.experimental import pallas as pl
from jax.experimental.pallas import tpu as pltpu
from jax.experimental.pallas import tpu_sc as plsc

K = 5
LOG31 = float(np.log(31.0))
LN2 = float(np.log(2.0))
NS = [12500, 3125, 780, 195]
ES = [200000, 50000, 12500, 3125]
CH = [(1, 32), (32, 64), (64, 128), (128, 256)]
N0 = 50000

NW = 32
CHUNK = 128
SWEEP = NW * CHUNK


def _cdiv(a, b):
    return (a + b - 1) // b


def _pad_to(v, m):
    return _cdiv(v, m) * m


EPAD = [_pad_to(e, SWEEP) for e in ES]
NPAD = [_pad_to(n, CHUNK) for n in NS]


def _log1p30(absd):
    z = 1.0 + 30.0 * absd
    bits = lax.bitcast_convert_type(z, jnp.int32)
    k = lax.shift_right_logical(bits, 23) - 127
    m = lax.bitcast_convert_type(
        lax.bitwise_or(lax.bitwise_and(bits, 0x007FFFFF), 0x3F800000),
        jnp.float32)
    t = (m - 1.0) / (m + 1.0)
    t2 = t * t
    lnm = t * (2.0 + t2 * (2.0 / 3.0 + t2 * (2.0 / 5.0 + t2 * (2.0 / 7.0 + t2 * (2.0 / 9.0)))))
    return k.astype(jnp.float32) * LN2 + lnm


def _basis(dx, dy, dz):
    s0 = jnp.float32(1.0)
    s1 = jnp.float32(1.0)
    for d in (dx, dy, dz):
        sgn = jnp.where(d < 0.0, -1.0, jnp.where(d > 0.0, 1.0, 0.0))
        u = 0.5 + 0.5 * sgn * _log1p30(jnp.abs(d)) / LOG31
        u = jnp.minimum(jnp.maximum(u, 0.0), 1.0)
        t = u * (K - 1)
        fl = t.astype(jnp.int32).astype(jnp.float32)
        frac = t - fl
        s0 = s0 * (1.0 - frac)
        s1 = s1 * frac
    return s0 + s1


def _iota16():
    return lax.iota(jnp.int32, 16)


def _seg_accum(keys, vals, acc_refs, kbuf, vbuf, ops=None):
    if ops is None:
        ops = ["add"] * len(vals)
    lanes = _iota16()
    sk, perm = plsc.sort_key_val(keys, lanes)
    kbuf[...] = sk
    pvals = []
    for v in vals:
        vbuf[...] = v
        pvals.append(plsc.load_gather(vbuf, [perm]))
    for st in (1, 2, 4, 8):
        idx = jnp.maximum(lanes - st, 0)
        kb = plsc.load_gather(kbuf, [idx])
        samek = jnp.logical_and(kb == sk, lanes >= st)
        for j, op in enumerate(ops):
            vbuf[...] = pvals[j]
            vb = plsc.load_gather(vbuf, [idx])
            if op == "add":
                pvals[j] = pvals[j] + jnp.where(samek, vb, 0.0)
            else:
                pvals[j] = jnp.maximum(pvals[j], jnp.where(samek, vb, -jnp.inf))
    knext = plsc.load_gather(kbuf, [jnp.minimum(lanes + 1, 15)])
    last = jnp.logical_or(lanes == 15, knext != sk)
    for ref, v, op in zip(acc_refs, pvals, ops):
        if op == "add":
            plsc.addupdate_scatter(ref, [sk], v, mask=last)
        else:
            old = plsc.load_gather(ref, [sk])
            plsc.store_scatter(ref, [sk], jnp.maximum(old, v), mask=last)


PNP = [16, 16, 16, 16]
NINPAD = [_pad_to(v, PNP[i] * CHUNK) for i, v in enumerate((N0, NS[0], NS[1], NS[2]))]


def _pool_l0(cl, x, px, py, pz):
    npad = NPAD[0]
    NP = PNP[0]
    cpt = NINPAD[0] // (NP * CHUNK)
    mesh = plsc.VectorSubcoreMesh(core_axis_name="c", subcore_axis_name="s")

    @functools.partial(
        pl.kernel, mesh=mesh,
        compiler_params=pltpu.CompilerParams(needs_layout_passes=False),
        out_type=[jax.ShapeDtypeStruct((NP, npad), jnp.float32)
                  for _ in range(5)],
        scratch_types=[
            pltpu.VMEM((CHUNK,), jnp.int32),
            pltpu.VMEM((CHUNK,), jnp.float32),
            pltpu.VMEM((CHUNK,), jnp.float32),
            pltpu.VMEM((CHUNK,), jnp.float32),
            pltpu.VMEM((CHUNK,), jnp.float32),
            pltpu.VMEM((npad,), jnp.float32),
            pltpu.VMEM((npad,), jnp.float32),
            pltpu.VMEM((npad,), jnp.float32),
            pltpu.VMEM((npad,), jnp.float32),
            pltpu.VMEM((npad,), jnp.float32),
            pltpu.VMEM((16,), jnp.int32),
            pltpu.VMEM((16,), jnp.float32),
        ])
    def body(cl_h, x_h, px_h, py_h, pz_h, xm_h, sx_h, sy_h, sz_h, ct_h,
             cl_v, x_v, px_v, py_v, pz_v, am, ax, ay, az, ac, kbuf, vbuf):
        cid = lax.axis_index("c")
        sid = lax.axis_index("s")
        wid = sid * 2 + cid

        @pl.when(wid < NP)
        def _():
            zero16 = jnp.zeros((16,), jnp.float32)
            ninf16 = jnp.full((16,), -jnp.inf, jnp.float32)
            def zacc(r, _):
                sl = pl.ds(r * 16, 16)
                am[sl] = ninf16
                ax[sl] = zero16
                ay[sl] = zero16
                az[sl] = zero16
                ac[sl] = zero16
                return 0
            lax.fori_loop(0, npad // 16, zacc, 0)

            def chunk(j, _):
                base = (j * NP + wid) * CHUNK
                pltpu.sync_copy(cl_h.at[pl.ds(base, CHUNK)], cl_v)
                pltpu.sync_copy(x_h.at[pl.ds(base, CHUNK)], x_v)
                pltpu.sync_copy(px_h.at[pl.ds(base, CHUNK)], px_v)
                pltpu.sync_copy(py_h.at[pl.ds(base, CHUNK)], py_v)
                pltpu.sync_copy(pz_h.at[pl.ds(base, CHUNK)], pz_v)
                for g in range(CHUNK // 16):
                    sl = pl.ds(g * 16, 16)
                    keys = cl_v[sl]
                    one = jnp.ones((16,), jnp.float32)
                    _seg_accum(keys, [x_v[sl], px_v[sl], py_v[sl], pz_v[sl], one],
                               [am, ax, ay, az, ac], kbuf, vbuf,
                               ops=["max", "add", "add", "add", "add"])
                return 0
            lax.fori_loop(0, cpt, chunk, 0)
            pltpu.sync_copy(am, xm_h.at[wid])
            pltpu.sync_copy(ax, sx_h.at[wid])
            pltpu.sync_copy(ay, sy_h.at[wid])
            pltpu.sync_copy(az, sz_h.at[wid])
            pltpu.sync_copy(ac, ct_h.at[wid])

    return body(cl, x, px, py, pz)


def _pool_lvl(i, cl, x, px, py, pz):
    npad = NPAD[i]
    cin = CH[i][0]
    NP = PNP[i]
    cpt = NINPAD[i] // (NP * CHUNK)
    mesh = plsc.VectorSubcoreMesh(core_axis_name="c", subcore_axis_name="s")

    @functools.partial(
        pl.kernel, mesh=mesh,
        compiler_params=pltpu.CompilerParams(needs_layout_passes=False),
        out_type=[jax.ShapeDtypeStruct((NP, npad * cin), jnp.float32)]
                 + [jax.ShapeDtypeStruct((NP, npad), jnp.float32)
                    for _ in range(4)],
        scratch_types=[
            pltpu.VMEM((CHUNK,), jnp.int32),
            pltpu.VMEM((CHUNK * cin,), jnp.float32),
            pltpu.VMEM((CHUNK,), jnp.float32),
            pltpu.VMEM((CHUNK,), jnp.float32),
            pltpu.VMEM((CHUNK,), jnp.float32),
            pltpu.VMEM((npad * cin,), jnp.float32),
            pltpu.VMEM((npad,), jnp.float32),
            pltpu.VMEM((npad,), jnp.float32),
            pltpu.VMEM((npad,), jnp.float32),
            pltpu.VMEM((npad,), jnp.float32),
            pltpu.VMEM((16,), jnp.int32),
            pltpu.VMEM((16,), jnp.float32),
        ])
    def body(cl_h, x_h, px_h, py_h, pz_h, xm_h, sx_h, sy_h, sz_h, ct_h,
             cl_v, xr_v, px_v, py_v, pz_v, am, ax, ay, az, ac, kbuf, vbuf):
        cid = lax.axis_index("c")
        sid = lax.axis_index("s")
        wid = sid * 2 + cid

        @pl.when(wid < NP)
        def _():
            zero16 = jnp.zeros((16,), jnp.float32)
            ninf16 = jnp.full((16,), -jnp.inf, jnp.float32)
            def zacc(r, _):
                am[pl.ds(r * 16, 16)] = ninf16
                return 0
            lax.fori_loop(0, npad * cin // 16, zacc, 0)
            def zacc2(r, _):
                sl = pl.ds(r * 16, 16)
                ax[sl] = zero16
                ay[sl] = zero16
                az[sl] = zero16
                ac[sl] = zero16
                return 0
            lax.fori_loop(0, npad // 16, zacc2, 0)

            def chunk(j, _):
                base = (j * NP + wid) * CHUNK
                pltpu.sync_copy(cl_h.at[pl.ds(base, CHUNK)], cl_v)
                pltpu.sync_copy(x_h.at[pl.ds(base * cin, CHUNK * cin)], xr_v)
                pltpu.sync_copy(px_h.at[pl.ds(base, CHUNK)], px_v)
                pltpu.sync_copy(py_h.at[pl.ds(base, CHUNK)], py_v)
                pltpu.sync_copy(pz_h.at[pl.ds(base, CHUNK)], pz_v)
                for g in range(CHUNK // 16):
                    sl = pl.ds(g * 16, 16)
                    keys = cl_v[sl]
                    one = jnp.ones((16,), jnp.float32)
                    _seg_accum(keys, [px_v[sl], py_v[sl], pz_v[sl], one],
                               [ax, ay, az, ac], kbuf, vbuf)
                    for lane in range(16):
                        kb = keys[lane] * cin
                        rb = (g * 16 + lane) * cin
                        for c in range(cin // 16):
                            asl = pl.ds(kb + c * 16, 16)
                            xsl = pl.ds(rb + c * 16, 16)
                            am[asl] = jnp.maximum(am[asl], xr_v[xsl])
                return 0
            lax.fori_loop(0, cpt, chunk, 0)
            pltpu.sync_copy(am, xm_h.at[wid])
            pltpu.sync_copy(ax, sx_h.at[wid])
            pltpu.sync_copy(ay, sy_h.at[wid])
            pltpu.sync_copy(az, sz_h.at[wid])
            pltpu.sync_copy(ac, ct_h.at[wid])

    return body(cl, x, px, py, pz)


def _edge_l0(eidx, clpacked, xp, px, py, pz):
    n, e_true, e_pad, npad = NS[0], ES[0], EPAD[0], NPAD[0]
    nhalf = clpacked.shape[0]
    cpt = e_pad // SWEEP
    mesh = plsc.VectorSubcoreMesh(core_axis_name="c", subcore_axis_name="s")

    @functools.partial(
        pl.kernel, mesh=mesh,
        compiler_params=pltpu.CompilerParams(needs_layout_passes=False),
        out_type=[jax.ShapeDtypeStruct((NW, npad), jnp.float32),
                  jax.ShapeDtypeStruct((NW, npad), jnp.float32),
                  jax.ShapeDtypeStruct((2, e_pad), jnp.int32)],
        scratch_types=[
            pltpu.VMEM((nhalf,), jnp.int32),
            pltpu.VMEM((n,), jnp.float32),
            pltpu.VMEM((n,), jnp.float32),
            pltpu.VMEM((n,), jnp.float32),
            pltpu.VMEM((n,), jnp.float32),
            pltpu.VMEM((CHUNK,), jnp.int32),
            pltpu.VMEM((CHUNK,), jnp.int32),
            pltpu.VMEM((CHUNK,), jnp.int32),
            pltpu.VMEM((CHUNK,), jnp.int32),
            pltpu.VMEM((npad,), jnp.float32),
            pltpu.VMEM((npad,), jnp.float32),
            pltpu.VMEM((16,), jnp.int32),
            pltpu.VMEM((16,), jnp.float32),
        ])
    def body(eidx_h, cl_h, x_h, px_h, py_h, pz_h, asum_h, deg_h, emap_h,
             cl_v, x_v, px_v, py_v, pz_v, rs_v, rd_v, ms_v, md_v,
             acc_a, acc_d, kbuf, vbuf):
        cid = lax.axis_index("c")
        sid = lax.axis_index("s")
        wid = sid * 2 + cid
        pltpu.sync_copy(cl_h, cl_v)
        pltpu.sync_copy(x_h, x_v)
        pltpu.sync_copy(px_h, px_v)
        pltpu.sync_copy(py_h, py_v)
        pltpu.sync_copy(pz_h, pz_v)
        zero16 = jnp.zeros((16,), jnp.float32)
        def zacc(r, _):
            acc_a[pl.ds(r * 16, 16)] = zero16
            acc_d[pl.ds(r * 16, 16)] = zero16
            return 0
        lax.fori_loop(0, npad // 16, zacc, 0)

        def chunk(j, _):
            base = j * SWEEP + wid * CHUNK
            pltpu.sync_copy(eidx_h.at[0, pl.ds(base, CHUNK)], rs_v)
            pltpu.sync_copy(eidx_h.at[1, pl.ds(base, CHUNK)], rd_v)
            for g in range(CHUNK // 16):
                sl = pl.ds(g * 16, 16)
                raws = rs_v[sl]
                rawd = rd_v[sl]
                ws = plsc.load_gather(cl_v, [lax.shift_right_logical(raws, 1)])
                wd = plsc.load_gather(cl_v, [lax.shift_right_logical(rawd, 1)])
                sv = lax.bitwise_and(
                    lax.shift_right_logical(ws, lax.bitwise_and(raws, 1) * 16), 0xFFFF)
                dv = lax.bitwise_and(
                    lax.shift_right_logical(wd, lax.bitwise_and(rawd, 1) * 16), 0xFFFF)
                ms_v[sl] = sv
                md_v[sl] = dv
                dx = plsc.load_gather(px_v, [dv]) - plsc.load_gather(px_v, [sv])
                dy = plsc.load_gather(py_v, [dv]) - plsc.load_gather(py_v, [sv])
                dz = plsc.load_gather(pz_v, [dv]) - plsc.load_gather(pz_v, [sv])
                s = _basis(dx, dy, dz)
                validf = jnp.where(base + g * 16 + _iota16() < e_true, 1.0, 0.0)
                a = s * plsc.load_gather(x_v, [sv]) * validf
                _seg_accum(dv, [a, validf], [acc_a, acc_d], kbuf, vbuf)
            pltpu.sync_copy(ms_v, emap_h.at[0, pl.ds(base, CHUNK)])
            pltpu.sync_copy(md_v, emap_h.at[1, pl.ds(base, CHUNK)])
            return 0
        lax.fori_loop(0, cpt, chunk, 0)
        pltpu.sync_copy(acc_a, asum_h.at[wid])
        pltpu.sync_copy(acc_d, deg_h.at[wid])

    return body(eidx, clpacked, xp, px, py, pz)


def _edge_lvl(i, eprev, cl, h, px, py, pz):
    n, n_in, e_true, e_pad, npad = NS[i], NS[i - 1], ES[i], EPAD[i], NPAD[i]
    cout = CH[i][1]
    nrep = _cdiv(cout, 128)
    cpt = e_pad // SWEEP
    mesh = plsc.VectorSubcoreMesh(core_axis_name="c", subcore_axis_name="s")

    @functools.partial(
        pl.kernel, mesh=mesh,
        compiler_params=pltpu.CompilerParams(needs_layout_passes=False),
        out_type=[jax.ShapeDtypeStruct((2, nrep * npad, 128), jnp.float32),
                  jax.ShapeDtypeStruct((NW, npad), jnp.float32),
                  jax.ShapeDtypeStruct((2, e_pad), jnp.int32)],
        scratch_types=[
            pltpu.VMEM((n_in,), jnp.int32),
            pltpu.VMEM((n,), jnp.float32),
            pltpu.VMEM((n,), jnp.float32),
            pltpu.VMEM((n,), jnp.float32),
            pltpu.VMEM((CHUNK,), jnp.int32),
            pltpu.VMEM((CHUNK,), jnp.int32),
            pltpu.VMEM((CHUNK,), jnp.int32),
            pltpu.VMEM((CHUNK,), jnp.int32),
            pltpu.VMEM((CHUNK,), jnp.int32),
            pltpu.VMEM((CHUNK,), jnp.int32),
            pltpu.VMEM((CHUNK,), jnp.float32),
            pltpu.VMEM((CHUNK, 128), jnp.float32),
            pltpu.VMEM((npad,), jnp.float32),
            pltpu.VMEM((16,), jnp.int32),
            pltpu.VMEM((16,), jnp.float32),
            pltpu.VMEM_SHARED((nrep * npad, 128), jnp.float32),
            pltpu.SemaphoreType.DMA,
        ])
    def body(eprev_h, cl_h, h_h, px_h, py_h, pz_h, sums_h, deg_h, emap_h,
             cl_v, px_v, py_v, pz_v, rs_v, rd_v, ms_v, md_v, msj_v, mdj_v,
             s_v, rows_v, acc_d, kbuf, vbuf, acc_sh, sem):
        cid = lax.axis_index("c")
        sid = lax.axis_index("s")
        wid = sid * 2 + cid
        pltpu.sync_copy(cl_h, cl_v)
        pltpu.sync_copy(px_h, px_v)
        pltpu.sync_copy(py_h, py_v)
        pltpu.sync_copy(pz_h, pz_v)
        zero16 = jnp.zeros((16,), jnp.float32)
        def zrow(r, _):
            for c in range(128 // 16):
                rows_v[r, pl.ds(c * 16, 16)] = zero16
            return 0
        lax.fori_loop(0, CHUNK, zrow, 0)
        def zacc(r, _):
            acc_d[pl.ds(r * 16, 16)] = zero16
            return 0
        lax.fori_loop(0, npad // 16, zacc, 0)
        for rc in range(nrep * npad // CHUNK):
            @pl.when(sid == rc % 16)
            def _():
                pltpu.sync_copy(rows_v, acc_sh.at[pl.ds(rc * CHUNK, CHUNK)])
        plsc.subcore_barrier()

        def chunk(j, _):
            base = j * SWEEP + wid * CHUNK
            pltpu.sync_copy(eprev_h.at[0, pl.ds(base, CHUNK)], rs_v)
            pltpu.sync_copy(eprev_h.at[1, pl.ds(base, CHUNK)], rd_v)
            for g in range(CHUNK // 16):
                sl = pl.ds(g * 16, 16)
                sv = plsc.load_gather(cl_v, [rs_v[sl]])
                dv = plsc.load_gather(cl_v, [rd_v[sl]])
                ms_v[sl] = sv
                md_v[sl] = dv
                dx = plsc.load_gather(px_v, [dv]) - plsc.load_gather(px_v, [sv])
                dy = plsc.load_gather(py_v, [dv]) - plsc.load_gather(py_v, [sv])
                dz = plsc.load_gather(pz_v, [dv]) - plsc.load_gather(pz_v, [sv])
                s = _basis(dx, dy, dz)
                validf = jnp.where(base + g * 16 + _iota16() < e_true, 1.0, 0.0)
                s_v[sl] = s * validf
                _seg_accum(dv, [validf], [acc_d], kbuf, vbuf)
            for j in range(nrep):
                if nrep > 1:
                    def scl_idx(g, _):
                        sl = pl.ds(g * 16, 16)
                        msj_v[sl] = ms_v[sl] * nrep + j
                        mdj_v[sl] = md_v[sl] * nrep + j
                        return 0
                    lax.fori_loop(0, CHUNK // 16, scl_idx, 0)
                    src_idx, dst_idx = msj_v, mdj_v
                else:
                    src_idx, dst_idx = ms_v, md_v
                pltpu.async_copy(h_h.at[src_idx], rows_v, sem).wait()
                def scale(g, _):
                    sg = s_v[pl.ds(g * 16, 16)]
                    for lane in range(16):
                        sval = sg[lane]
                        r = g * 16 + lane
                        for c in range(128 // 16):
                            csl = pl.ds(c * 16, 16)
                            rows_v[r, csl] = rows_v[r, csl] * sval
                    return 0
                lax.fori_loop(0, CHUNK // 16, scale, 0)
                pltpu.sync_copy(rows_v, acc_sh.at[dst_idx], add=True)
            pltpu.sync_copy(ms_v, emap_h.at[0, pl.ds(base, CHUNK)])
            pltpu.sync_copy(md_v, emap_h.at[1, pl.ds(base, CHUNK)])
            return 0
        lax.fori_loop(0, cpt, chunk, 0)
        plsc.subcore_barrier()
        for rc in range(nrep * npad // CHUNK):
            @pl.when(sid == rc % 16)
            def _():
                pltpu.sync_copy(acc_sh.at[pl.ds(rc * CHUNK, CHUNK)],
                                sums_h.at[cid, pl.ds(rc * CHUNK, CHUNK)])
        pltpu.sync_copy(acc_d, deg_h.at[wid])

    return body(eprev, cl, h, px, py, pz)


def _elu(v):
    return jnp.where(v > 0, v, jnp.exp(jnp.minimum(v, 0.0)) - 1.0)


def _combine0_body(asum_ref, degs_ref, xp_ref, w_ref, r_ref, b_ref, out_ref):
    a = jnp.sum(asum_ref[...], axis=0)[:NS[0]]
    deg = jnp.sum(degs_ref[...], axis=0)[:NS[0]]
    agg = (a / jnp.maximum(deg, 1.0))[:, None] * w_ref[0][None, :]
    root = xp_ref[...] @ r_ref[...]
    out_ref[...] = _elu(agg + root + b_ref[...][None, :])


def _combine0(asum, degs, xp, W1, R1, b1):
    return pl.pallas_call(
        _combine0_body,
        out_shape=jax.ShapeDtypeStruct((NS[0], CH[0][1]), jnp.float32),
    )(asum, degs, xp, W1, R1, b1)


def _poolcomb0_body(xm_ref, sx_ref, sy_ref, sz_ref, ct_ref,
                    xp_ref, px_ref, py_ref, pz_ref):
    n = NS[0]
    xm = jnp.max(xm_ref[...], axis=0)[:n]
    xp_ref[...] = jnp.where(jnp.isfinite(xm), xm, 0.0)[:, None]
    cnt = jnp.maximum(jnp.sum(ct_ref[...], axis=0)[:n], 1.0)
    px_ref[...] = jnp.sum(sx_ref[...], axis=0)[:n] / cnt
    py_ref[...] = jnp.sum(sy_ref[...], axis=0)[:n] / cnt
    pz_ref[...] = jnp.sum(sz_ref[...], axis=0)[:n] / cnt


def _poolcomb0(xm, sx, sy, sz, ct):
    n = NS[0]
    return pl.pallas_call(
        _poolcomb0_body,
        out_shape=[jax.ShapeDtypeStruct((n, 1), jnp.float32)]
                  + [jax.ShapeDtypeStruct((n,), jnp.float32)] * 3,
    )(xm, sx, sy, sz, ct)


def _prep_pool_body(i, xm_ref, sx_ref, sy_ref, sz_ref, ct_ref,
                    w_ref, rw_ref, b_ref,
                    h_ref, r_ref, px_ref, py_ref, pz_ref):
    n, cout = NS[i], CH[i][1]
    cin = CH[i][0]
    nrep = _cdiv(cout, 128)
    xm = jnp.max(xm_ref[...].reshape(PNP[i], -1, cin), axis=0)[:n]
    xp = jnp.where(jnp.isfinite(xm), xm, 0.0)
    cnt = jnp.maximum(jnp.sum(ct_ref[...], axis=0)[:n], 1.0)
    px_ref[...] = jnp.sum(sx_ref[...], axis=0)[:n] / cnt
    py_ref[...] = jnp.sum(sy_ref[...], axis=0)[:n] / cnt
    pz_ref[...] = jnp.sum(sz_ref[...], axis=0)[:n] / cnt
    h = xp @ w_ref[...]
    if cout < 128:
        h = jnp.concatenate(
            [h, jnp.zeros((n, 128 - cout), jnp.float32)], axis=1)
    h_ref[...] = h.reshape(nrep * n, 128)
    r_ref[...] = xp @ rw_ref[...] + b_ref[...][None, :]


def _prep_pool(i, xm, sx, sy, sz, ct, W, R, b):
    n, cout = NS[i], CH[i][1]
    nrep = _cdiv(cout, 128)
    return pl.pallas_call(
        functools.partial(_prep_pool_body, i),
        out_shape=[jax.ShapeDtypeStruct((nrep * n, 128), jnp.float32),
                   jax.ShapeDtypeStruct((n, cout), jnp.float32)]
                  + [jax.ShapeDtypeStruct((n,), jnp.float32)] * 3,
    )(xm, sx, sy, sz, ct, W, R, b)


def _combine_body(n, cout, npad, sums_ref, degs_ref, r_ref, out_ref):
    nrep = _cdiv(cout, 128)
    acc = (sums_ref[0] + sums_ref[1]).reshape(npad, nrep * 128)
    deg = jnp.sum(degs_ref[...], axis=0)
    agg = acc[:n, :cout] / jnp.maximum(deg[:n], 1.0)[:, None]
    out_ref[...] = _elu(agg + r_ref[...])


def _combine(i, sums, degs, r):
    n, cout = NS[i], CH[i][1]
    return pl.pallas_call(
        functools.partial(_combine_body, n, cout, NPAD[i]),
        out_shape=jax.ShapeDtypeStruct((n, cout), jnp.float32),
    )(sums, degs, r)


def _head_body(x_ref, pos_ref, fc1w_ref, fc1b_ref, fc2w_ref, fc2b_ref, out_ref):
    pos = pos_ref[...]
    vid = jnp.clip(jnp.floor(pos * 2.0), 0, 1).astype(jnp.int32)
    vox = vid[:, 0] * 4 + vid[:, 1] * 2 + vid[:, 2]
    x = x_ref[...]
    cells = []
    for k in range(8):
        m = (vox == k)[:, None]
        cells.append(jnp.max(jnp.where(m, x, -jnp.inf), axis=0))
    xv = jnp.stack(cells, axis=0)
    xv = jnp.where(jnp.isfinite(xv), xv, 0.0)
    hidden = xv.reshape(1, 8 * 256) @ fc1w_ref[...] + fc1b_ref[...][None, :]
    hidden = _elu(hidden)
    o = hidden @ fc2w_ref[...] + fc2b_ref[...][None, :]
    out_ref[...] = jax.nn.log_softmax(o, axis=1)


def _head(x, pos, fc1_w, fc1_b, fc2_w, fc2_b):
    return pl.pallas_call(
        _head_body,
        out_shape=jax.ShapeDtypeStruct((1, 10), jnp.float32),
    )(x, pos, fc1_w, fc1_b, fc2_w, fc2_b)


def _padv(v, ln, val):
    return jnp.concatenate([v, jnp.full((ln - v.shape[0],), val, v.dtype)])


def _padr(m, ln):
    return jnp.concatenate(
        [m, jnp.zeros((ln - m.shape[0], m.shape[1]), m.dtype)], axis=0)


def kernel(x, pos, edge_index, cluster1, cluster2, cluster3, cluster4,
           W1, R1, b1, W2, R2, b2, W3, R3, b3, W4, R4, b4,
           fc1_w, fc1_b, fc2_w, fc2_b):
    clusters = [cluster1, cluster2, cluster3, cluster4]
    Ws = [(W1, R1, b1), (W2, R2, b2), (W3, R3, b3), (W4, R4, b4)]
    e = edge_index

    ninp = NINPAD[0]
    cl0p = _padv(cluster1, ninp, NPAD[0] - 1)
    x0p = _padv(x[:, 0], ninp, 0.0)
    p0 = [_padv(pos[:, d], ninp, 0.0) for d in range(3)]
    xm, sx, sy, sz, ct = _pool_l0(cl0p, x0p, *p0)
    xp1, px, py, pz = _poolcomb0(xm, sx, sy, sz, ct)
    cu = cluster1.astype(jnp.uint32)
    clp = (cu[0::2] | (cu[1::2] << 16)).astype(jnp.int32)
    asum, degs, e = _edge_l0(e, clp, jnp.reshape(xp1, (NS[0],)), px, py, pz)
    xcur = _combine0(asum, degs, xp1, *Ws[0])

    for i in range(1, 4):
        c = clusters[i]
        ninp = NINPAD[i]
        clip_ = _padv(c, ninp, NPAD[i] - 1)
        xpad = _padr(xcur, ninp).reshape(-1)
        ppad = [_padv(q, ninp, 0.0) for q in (px, py, pz)]
        xm, sx, sy, sz, ct = _pool_lvl(i, clip_, xpad, *ppad)
        W, R, b = Ws[i]
        h, r, px, py, pz = _prep_pool(i, xm, sx, sy, sz, ct, W, R, b)
        sums, degs, e = _edge_lvl(i, e, c, h, px, py, pz)
        xcur = _combine(i, sums, degs, r)

    pos4 = jnp.stack([px, py, pz], axis=1)
    return _head(xcur, pos4, fc1_w, fc1_b, fc2_w, fc2_b)
